# Initial kernel scaffold; baseline (speedup 1.0000x reference)
import jax
import jax.numpy as jnp
from jax.experimental import pallas as pl


def kernel(x, fg_w1, fg_b1, fg_w2, fg_b2, expert_emb, qp_w, qp_b, out_w, out_b, gnn_w0, gnn_b0, gnn_g0, gnn_beta0, gnn_w1, gnn_b1, gnn_g1, gnn_beta1):
    raise NotImplementedError("write your pallas kernel here")


# speedup vs baseline: 3.4968x; 3.4968x over previous
import functools

import jax
import jax.numpy as jnp
from jax.experimental import pallas as pl

HIDDEN = 2048
H2 = HIDDEN // 2
E = 64
G = 256
CONF_THRESH = 0.75
EPS = 1e-08
LN_EPS = 1e-05

TB = 64
B = 8192


def _gelu(v):
    return 0.5 * v * (1.0 + jax.lax.erf(v * 0.7071067811865476))


def _layernorm(v, g, b):
    mu = jnp.mean(v, axis=-1, keepdims=True)
    var = jnp.mean((v - mu) * (v - mu), axis=-1, keepdims=True)
    return (v - mu) / jnp.sqrt(var + LN_EPS) * g + b


def _body(xb, w1t, b1, w2t, b2, qpt, qb, emb,
          w0nt, w0qt, b0, g0, be0,
          w1nt, w1qt, b1g, g1, be1,
          ow, ob, probs_ref, conf_ref):
    x = xb[...]
    h = _gelu(jnp.dot(x, w1t[...], preferred_element_type=jnp.float32) + b1[...])
    logits = jnp.dot(h, w2t[...], preferred_element_type=jnp.float32) + b2[...]
    m = jnp.max(logits, axis=-1, keepdims=True)
    ex = jnp.exp(logits - m)
    fast = ex / jnp.sum(ex, axis=-1, keepdims=True)
    conf = jnp.max(fast, axis=-1, keepdims=True)

    q = jnp.dot(x, qpt[...], preferred_element_type=jnp.float32) + qb[...]
    embv = emb[...]
    e0 = jnp.dot(embv, w0nt[...], preferred_element_type=jnp.float32)
    q0 = jnp.dot(q, w0qt[...], preferred_element_type=jnp.float32) + b0[...]
    pre0 = q0[:, None, :] + e0[None, :, :]
    ln0 = _layernorm(_gelu(pre0), g0[...], be0[...])
    nf1 = ln0 + embv[None, :, :]
    q1 = jnp.dot(q, w1qt[...], preferred_element_type=jnp.float32) + b1g[...]
    n1 = jnp.dot(nf1.reshape(TB * E, G), w1nt[...],
                 preferred_element_type=jnp.float32).reshape(TB, E, G)
    pre1 = n1 + q1[:, None, :]
    ln1 = _layernorm(_gelu(pre1), g1[...], be1[...])
    nf2 = ln1 + nf1
    scores = jnp.sum(nf2 * ow[...], axis=-1) + ob[0, 0]
    sm = jnp.max(scores, axis=-1, keepdims=True)
    es = jnp.exp(scores - sm)
    slow = es / jnp.sum(es, axis=-1, keepdims=True)

    blend = jax.nn.sigmoid((conf - CONF_THRESH) * 10.0)
    fp = blend * fast + (1.0 - blend) * slow
    fp = fp / (jnp.sum(fp, axis=-1, keepdims=True) + EPS)
    probs_ref[...] = fp
    conf_ref[...] = conf


def kernel(x, fg_w1, fg_b1, fg_w2, fg_b2, expert_emb, qp_w, qp_b, out_w, out_b,
           gnn_w0, gnn_b0, gnn_g0, gnn_beta0, gnn_w1, gnn_b1, gnn_g1, gnn_beta1):
    nb = B // TB
    row = lambda v: v.reshape(1, -1)
    w1t = fg_w1.T
    w2t = fg_w2.T
    qpt = qp_w.T
    w0nt = gnn_w0[:, :G].T
    w0qt = gnn_w0[:, G:].T
    w1nt = gnn_w1[:, :G].T
    w1qt = gnn_w1[:, G:].T

    tok = lambda i: (i, 0)
    rep = lambda i: (0, 0)

    probs, conf = pl.pallas_call(
        _body,
        grid=(nb,),
        in_specs=[
            pl.BlockSpec((TB, HIDDEN), tok),
            pl.BlockSpec((HIDDEN, H2), rep),
            pl.BlockSpec((1, H2), rep),
            pl.BlockSpec((H2, E), rep),
            pl.BlockSpec((1, E), rep),
            pl.BlockSpec((HIDDEN, G), rep),
            pl.BlockSpec((1, G), rep),
            pl.BlockSpec((E, G), rep),
            pl.BlockSpec((G, G), rep),
            pl.BlockSpec((G, G), rep),
            pl.BlockSpec((1, G), rep),
            pl.BlockSpec((1, G), rep),
            pl.BlockSpec((1, G), rep),
            pl.BlockSpec((G, G), rep),
            pl.BlockSpec((G, G), rep),
            pl.BlockSpec((1, G), rep),
            pl.BlockSpec((1, G), rep),
            pl.BlockSpec((1, G), rep),
            pl.BlockSpec((1, G), rep),
            pl.BlockSpec((1, 1), rep),
        ],
        out_specs=[
            pl.BlockSpec((TB, E), tok),
            pl.BlockSpec((TB, 1), tok),
        ],
        out_shape=[
            jax.ShapeDtypeStruct((B, E), jnp.float32),
            jax.ShapeDtypeStruct((B, 1), jnp.float32),
        ],
    )(
        x, w1t, row(fg_b1), w2t, row(fg_b2), qpt, row(qp_b), expert_emb,
        w0nt, w0qt, row(gnn_b0), row(gnn_g0), row(gnn_beta0),
        w1nt, w1qt, row(gnn_b1), row(gnn_g1), row(gnn_beta1),
        row(out_w), out_b.reshape(1, 1),
    )
    return (probs, conf.reshape(B))


# speedup vs baseline: 4.2905x; 1.2270x over previous
import functools

import jax
import jax.numpy as jnp
from jax.experimental import pallas as pl

HIDDEN = 2048
H2 = HIDDEN // 2
E = 64
G = 256
CONF_THRESH = 0.75
EPS = 1e-08
LN_EPS = 1e-05

TB = 128
B = 8192


def _gelu(v):
    return (0.5 * v) * (1.0 + jax.lax.erf(v * 0.7071067811865476))


def _stats(v):
    mu = jnp.mean(v, axis=-1, keepdims=True)
    ms = jnp.mean(v * v, axis=-1, keepdims=True)
    return mu, jax.lax.rsqrt(ms - mu * mu + LN_EPS)


def _body(xb, w1t, b1, w2t, b2, qpt, qb, emb,
          w0nt, wq, bq, be0,
          w1nt, g0, g1ow, be1ow,
          ow, ob, probs_ref, conf_ref):
    x = xb[...]
    h = _gelu(jnp.dot(x, w1t[...], preferred_element_type=jnp.float32) + b1[...])
    logits = jnp.dot(h, w2t[...], preferred_element_type=jnp.float32) + b2[...]
    m = jnp.max(logits, axis=-1, keepdims=True)
    ex = jnp.exp(logits - m)
    fast = ex / jnp.sum(ex, axis=-1, keepdims=True)
    conf = jnp.max(fast, axis=-1, keepdims=True)

    q = jnp.dot(x, qpt[...], preferred_element_type=jnp.float32) + qb[...]
    embv = emb[...]
    e0 = jnp.dot(embv, w0nt[...], preferred_element_type=jnp.float32)
    qq = jnp.dot(q, wq[...], preferred_element_type=jnp.float32) + bq[...]
    q0 = qq[:, :G]
    q1 = qq[:, G:]
    a0 = _gelu(q0[:, None, :] + e0[None, :, :])
    mu0, rsd0 = _stats(a0)
    nf1 = (a0 - mu0) * (rsd0 * g0[...]) + (be0[...] + embv)[None, :, :]
    s_nf1 = jnp.sum(nf1 * ow[...], axis=-1)
    n1 = jnp.dot(nf1.reshape(TB * E, G), w1nt[...],
                 preferred_element_type=jnp.float32).reshape(TB, E, G)
    a1 = _gelu(n1 + q1[:, None, :])
    mu1, rsd1 = _stats(a1)
    s_ln1 = jnp.sum((a1 - mu1) * (rsd1 * g1ow[...]), axis=-1)
    scores = s_ln1 + s_nf1 + (be1ow[0, 0] + ob[0, 0])
    sm = jnp.max(scores, axis=-1, keepdims=True)
    es = jnp.exp(scores - sm)
    slow = es / jnp.sum(es, axis=-1, keepdims=True)

    blend = jax.nn.sigmoid((conf - CONF_THRESH) * 10.0)
    fp = blend * fast + (1.0 - blend) * slow
    fp = fp / (jnp.sum(fp, axis=-1, keepdims=True) + EPS)
    probs_ref[...] = fp
    conf_ref[...] = conf


def kernel(x, fg_w1, fg_b1, fg_w2, fg_b2, expert_emb, qp_w, qp_b, out_w, out_b,
           gnn_w0, gnn_b0, gnn_g0, gnn_beta0, gnn_w1, gnn_b1, gnn_g1, gnn_beta1):
    nb = B // TB
    row = lambda v: v.reshape(1, -1)
    w1t = fg_w1.T
    w2t = fg_w2.T
    qpt = qp_w.T
    w0nt = gnn_w0[:, :G].T
    w0qt = gnn_w0[:, G:].T
    w1nt = gnn_w1[:, :G].T
    w1qt = gnn_w1[:, G:].T
    wq = jnp.concatenate([w0qt, w1qt], axis=1)
    bq = jnp.concatenate([gnn_b0, gnn_b1])
    ow = out_w[0]
    g1ow = gnn_g1 * ow
    be1ow = jnp.sum(gnn_beta1 * ow).reshape(1, 1)

    tok = lambda i: (i, 0)
    rep = lambda i: (0, 0)

    probs, conf = pl.pallas_call(
        _body,
        grid=(nb,),
        in_specs=[
            pl.BlockSpec((TB, HIDDEN), tok),
            pl.BlockSpec((HIDDEN, H2), rep),
            pl.BlockSpec((1, H2), rep),
            pl.BlockSpec((H2, E), rep),
            pl.BlockSpec((1, E), rep),
            pl.BlockSpec((HIDDEN, G), rep),
            pl.BlockSpec((1, G), rep),
            pl.BlockSpec((E, G), rep),
            pl.BlockSpec((G, G), rep),
            pl.BlockSpec((G, 2 * G), rep),
            pl.BlockSpec((1, 2 * G), rep),
            pl.BlockSpec((1, G), rep),
            pl.BlockSpec((G, G), rep),
            pl.BlockSpec((1, G), rep),
            pl.BlockSpec((1, G), rep),
            pl.BlockSpec((1, 1), rep),
            pl.BlockSpec((1, G), rep),
            pl.BlockSpec((1, 1), rep),
        ],
        out_specs=[
            pl.BlockSpec((TB, E), tok),
            pl.BlockSpec((TB, 1), tok),
        ],
        out_shape=[
            jax.ShapeDtypeStruct((B, E), jnp.float32),
            jax.ShapeDtypeStruct((B, 1), jnp.float32),
        ],
    )(
        x, w1t, row(fg_b1), w2t, row(fg_b2), qpt, row(qp_b), expert_emb,
        w0nt, wq, row(bq), row(gnn_beta0),
        w1nt, row(gnn_g0), row(g1ow), be1ow,
        row(ow), out_b.reshape(1, 1),
    )
    return (probs, conf.reshape(B))


# speedup vs baseline: 4.2910x; 1.0001x over previous
import functools

import jax
import jax.numpy as jnp
from jax.experimental import pallas as pl
from jax.experimental.pallas import tpu as pltpu

HIDDEN = 2048
H2 = HIDDEN // 2
E = 64
G = 256
CONF_THRESH = 0.75
EPS = 1e-08
LN_EPS = 1e-05

TB = 128
B = 8192


def _gelu(v):
    return (0.5 * v) * (1.0 + jax.lax.erf(v * 0.7071067811865476))


def _stats(v):
    mu = jnp.mean(v, axis=-1, keepdims=True)
    ms = jnp.mean(v * v, axis=-1, keepdims=True)
    return mu, jax.lax.rsqrt(ms - mu * mu + LN_EPS)


def _body(xb, w1t, b1, w2t, b2, qpt, qb, emb,
          w0nt, wq, bq, be0,
          w1nt, g0, g1ow, be1ow,
          ow, ob, probs_ref, conf_ref):
    x = xb[...]
    h = _gelu(jnp.dot(x, w1t[...], preferred_element_type=jnp.float32) + b1[...])
    logits = jnp.dot(h, w2t[...], preferred_element_type=jnp.float32) + b2[...]
    m = jnp.max(logits, axis=-1, keepdims=True)
    ex = jnp.exp(logits - m)
    fast = ex / jnp.sum(ex, axis=-1, keepdims=True)
    conf = jnp.max(fast, axis=-1, keepdims=True)

    q = jnp.dot(x, qpt[...], preferred_element_type=jnp.float32) + qb[...]
    embv = emb[...]
    e0 = jnp.dot(embv, w0nt[...], preferred_element_type=jnp.float32)
    qq = jnp.dot(q, wq[...], preferred_element_type=jnp.float32) + bq[...]
    q0 = qq[:, :G]
    q1 = qq[:, G:]
    a0 = _gelu(q0[:, None, :] + e0[None, :, :])
    mu0, rsd0 = _stats(a0)
    nf1 = (a0 - mu0) * (rsd0 * g0[...]) + (be0[...] + embv)[None, :, :]
    s_nf1 = jnp.sum(nf1 * ow[...], axis=-1)
    n1 = jnp.dot(nf1.reshape(TB * E, G), w1nt[...],
                 preferred_element_type=jnp.float32).reshape(TB, E, G)
    a1 = _gelu(n1 + q1[:, None, :])
    mu1, rsd1 = _stats(a1)
    s_ln1 = jnp.sum((a1 - mu1) * (rsd1 * g1ow[...]), axis=-1)
    scores = s_ln1 + s_nf1 + (be1ow[0, 0] + ob[0, 0])
    sm = jnp.max(scores, axis=-1, keepdims=True)
    es = jnp.exp(scores - sm)
    slow = es / jnp.sum(es, axis=-1, keepdims=True)

    blend = jax.nn.sigmoid((conf - CONF_THRESH) * 10.0)
    fp = blend * fast + (1.0 - blend) * slow
    fp = fp / (jnp.sum(fp, axis=-1, keepdims=True) + EPS)
    probs_ref[...] = fp
    conf_ref[...] = conf


def kernel(x, fg_w1, fg_b1, fg_w2, fg_b2, expert_emb, qp_w, qp_b, out_w, out_b,
           gnn_w0, gnn_b0, gnn_g0, gnn_beta0, gnn_w1, gnn_b1, gnn_g1, gnn_beta1):
    nb = B // TB
    row = lambda v: v.reshape(1, -1)
    w1t = fg_w1.T
    w2t = fg_w2.T
    qpt = qp_w.T
    w0nt = gnn_w0[:, :G].T
    w0qt = gnn_w0[:, G:].T
    w1nt = gnn_w1[:, :G].T
    w1qt = gnn_w1[:, G:].T
    wq = jnp.concatenate([w0qt, w1qt], axis=1)
    bq = jnp.concatenate([gnn_b0, gnn_b1])
    ow = out_w[0]
    g1ow = gnn_g1 * ow
    be1ow = jnp.sum(gnn_beta1 * ow).reshape(1, 1)

    tok = lambda i: (i, 0)
    rep = lambda i: (0, 0)

    probs, conf = pl.pallas_call(
        _body,
        grid=(nb,),
        compiler_params=pltpu.CompilerParams(
            dimension_semantics=("parallel",)),
        in_specs=[
            pl.BlockSpec((TB, HIDDEN), tok),
            pl.BlockSpec((HIDDEN, H2), rep),
            pl.BlockSpec((1, H2), rep),
            pl.BlockSpec((H2, E), rep),
            pl.BlockSpec((1, E), rep),
            pl.BlockSpec((HIDDEN, G), rep),
            pl.BlockSpec((1, G), rep),
            pl.BlockSpec((E, G), rep),
            pl.BlockSpec((G, G), rep),
            pl.BlockSpec((G, 2 * G), rep),
            pl.BlockSpec((1, 2 * G), rep),
            pl.BlockSpec((1, G), rep),
            pl.BlockSpec((G, G), rep),
            pl.BlockSpec((1, G), rep),
            pl.BlockSpec((1, G), rep),
            pl.BlockSpec((1, 1), rep),
            pl.BlockSpec((1, G), rep),
            pl.BlockSpec((1, 1), rep),
        ],
        out_specs=[
            pl.BlockSpec((TB, E), tok),
            pl.BlockSpec((TB, 1), tok),
        ],
        out_shape=[
            jax.ShapeDtypeStruct((B, E), jnp.float32),
            jax.ShapeDtypeStruct((B, 1), jnp.float32),
        ],
    )(
        x, w1t, row(fg_b1), w2t, row(fg_b2), qpt, row(qp_b), expert_emb,
        w0nt, wq, row(bq), row(gnn_beta0),
        w1nt, row(gnn_g0), row(g1ow), be1ow,
        row(ow), out_b.reshape(1, 1),
    )
    return (probs, conf.reshape(B))


# speedup vs baseline: 5.4510x; 1.2703x over previous
import functools

import jax
import jax.numpy as jnp
from jax.experimental import pallas as pl
from jax.experimental.pallas import tpu as pltpu

HIDDEN = 2048
H2 = HIDDEN // 2
E = 64
G = 256
CONF_THRESH = 0.75
EPS = 1e-08
LN_EPS = 1e-05

TB = 128
B = 8192


SQRT1_2 = 0.7071067811865476


def _gelu_pre(t):
    return (SQRT1_2 * t) * (1.0 + jax.lax.erf(t))


INV_G = 1.0 / G
SQRT_G = 16.0


def _fold(v):
    return v[..., :G // 2] + v[..., G // 2:]


def _stats(v):
    s = jnp.sum(_fold(v), axis=-1, keepdims=True)
    q = jnp.sum(_fold(v * v), axis=-1, keepdims=True)
    mu = s * INV_G
    r = jax.lax.rsqrt(q - s * mu + G * LN_EPS)
    return mu, r


def _body(xb, w1t, b1, w2t, b2, qpt, qb, emb,
          w0nt, wq, bq, be0,
          w1nt, g0, g1ow, be1ow,
          ow, ob, probs_ref, conf_ref, logits_s, scores_s):
    lg = logits_s[...]
    sc_prev = scores_s[...]
    m = jnp.max(lg, axis=-1, keepdims=True)
    ex = jnp.exp(lg - m)
    fast = ex / jnp.sum(ex, axis=-1, keepdims=True)
    conf = jnp.max(fast, axis=-1, keepdims=True)
    sm = jnp.max(sc_prev, axis=-1, keepdims=True)
    es = jnp.exp(sc_prev - sm)
    slow = es / jnp.sum(es, axis=-1, keepdims=True)
    blend = jax.nn.sigmoid((conf - CONF_THRESH) * 10.0)
    fp = blend * fast + (1.0 - blend) * slow
    fp = fp / (jnp.sum(fp, axis=-1, keepdims=True) + EPS)
    probs_ref[...] = fp
    conf_ref[...] = conf

    x = xb[...]
    h = _gelu_pre(jnp.dot(x, w1t[...], preferred_element_type=jnp.float32) + b1[...])
    logits = jnp.dot(h, w2t[...], preferred_element_type=jnp.float32) + b2[...]

    q = jnp.dot(x, qpt[...], preferred_element_type=jnp.float32) + qb[...]
    embv = emb[...]
    e0 = jnp.dot(embv, w0nt[...], preferred_element_type=jnp.float32)
    qq = jnp.dot(q, wq[...], preferred_element_type=jnp.float32) + bq[...]
    q0 = qq[:, :G]
    q1 = qq[:, G:]
    a0 = _gelu_pre(q0[:, None, :] + e0[None, :, :])
    mu0, rsd0 = _stats(a0)
    nf1 = (a0 - mu0) * (rsd0 * g0[...]) + (be0[...] + embv)[None, :, :]
    s_nf1 = jnp.sum(_fold(nf1 * ow[...]), axis=-1)
    n1 = jnp.dot(nf1.reshape(TB * E, G), w1nt[...],
                 preferred_element_type=jnp.float32).reshape(TB, E, G)
    a1 = _gelu_pre(n1 + q1[:, None, :])
    mu1, rsd1 = _stats(a1)
    s_ln1 = jnp.sum(_fold((a1 - mu1) * (rsd1 * g1ow[...])), axis=-1)
    scores = s_ln1 + s_nf1 + (be1ow[0, 0] + ob[0, 0])
    logits_s[...] = logits
    scores_s[...] = scores


def kernel(x, fg_w1, fg_b1, fg_w2, fg_b2, expert_emb, qp_w, qp_b, out_w, out_b,
           gnn_w0, gnn_b0, gnn_g0, gnn_beta0, gnn_w1, gnn_b1, gnn_g1, gnn_beta1):
    nb = B // TB
    row = lambda v: v.reshape(1, -1)
    c = SQRT1_2
    w1t = fg_w1.T * c
    w2t = fg_w2.T
    qpt = qp_w.T
    w0nt = gnn_w0[:, :G].T * c
    w0qt = gnn_w0[:, G:].T * c
    w1nt = gnn_w1[:, :G].T * c
    w1qt = gnn_w1[:, G:].T * c
    wq = jnp.concatenate([w0qt, w1qt], axis=1)
    bq = jnp.concatenate([gnn_b0, gnn_b1]) * c
    ow = out_w[0]
    g0s = gnn_g0 * SQRT_G
    g1ow = gnn_g1 * ow * SQRT_G
    be1ow = jnp.sum(gnn_beta1 * ow).reshape(1, 1)

    tok = lambda i: (jnp.minimum(i, nb - 1), 0)
    out = lambda i: (jnp.maximum(i - 1, 0), 0)
    rep = lambda i: (0, 0)

    probs, conf = pl.pallas_call(
        _body,
        grid=(nb + 1,),
        scratch_shapes=[
            pltpu.VMEM((TB, E), jnp.float32),
            pltpu.VMEM((TB, E), jnp.float32),
        ],
        in_specs=[
            pl.BlockSpec((TB, HIDDEN), tok),
            pl.BlockSpec((HIDDEN, H2), rep),
            pl.BlockSpec((1, H2), rep),
            pl.BlockSpec((H2, E), rep),
            pl.BlockSpec((1, E), rep),
            pl.BlockSpec((HIDDEN, G), rep),
            pl.BlockSpec((1, G), rep),
            pl.BlockSpec((E, G), rep),
            pl.BlockSpec((G, G), rep),
            pl.BlockSpec((G, 2 * G), rep),
            pl.BlockSpec((1, 2 * G), rep),
            pl.BlockSpec((1, G), rep),
            pl.BlockSpec((G, G), rep),
            pl.BlockSpec((1, G), rep),
            pl.BlockSpec((1, G), rep),
            pl.BlockSpec((1, 1), rep),
            pl.BlockSpec((1, G), rep),
            pl.BlockSpec((1, 1), rep),
        ],
        out_specs=[
            pl.BlockSpec((TB, E), out),
            pl.BlockSpec((TB, 1), out),
        ],
        out_shape=[
            jax.ShapeDtypeStruct((B, E), jnp.float32),
            jax.ShapeDtypeStruct((B, 1), jnp.float32),
        ],
    )(
        x, w1t, row(fg_b1 * c), w2t, row(fg_b2), qpt, row(qp_b), expert_emb,
        w0nt, wq, row(bq), row(gnn_beta0),
        w1nt, row(g0s), row(g1ow), be1ow,
        row(ow), out_b.reshape(1, 1),
    )
    return (probs, conf.reshape(B))
